# baseline (speedup 1.0000x reference)
import functools

import jax
import jax.numpy as jnp
from jax import lax
from jax.experimental import pallas as pl
from jax.experimental.pallas import tpu as pltpu
from jax.experimental.pallas import tpu_sc as plsc

N_CODES = 1024
EDIM = 64
BETA = 0.25


def _dist_kernel(z_ref, emb_ref, zpsq_ref, embsq_ref,
                 idx_ref, loss_ref, acc_ref):
    i = pl.program_id(0)
    nsteps = pl.num_programs(0)
    z_n = z_ref[0]
    emb = emb_ref[...]
    T = z_n.shape[1]
    K = emb.shape[0]

    zp = z_n.T

    dot = jax.lax.dot_general(zp, emb, (((1,), (1,)), ((), ())))
    d = (zpsq_ref[...] + embsq_ref[...]) - 2.0 * dot

    dmin = jnp.min(d, axis=1, keepdims=True)
    iota_k = jax.lax.broadcasted_iota(jnp.int32, (T, K), 1)
    idx = jnp.min(jnp.where(d == dmin, iota_k, K), axis=1)
    idx_ref[0] = idx[None, :]

    part = jnp.sum(dmin)

    @pl.when(i == 0)
    def _():
        acc_ref[0, 0] = 0.0

    acc_ref[0, 0] += part

    @pl.when(i == nsteps - 1)
    def _():
        m = acc_ref[0, 0] / (nsteps * T * EDIM)
        loss_ref[...] = jnp.reshape(m + BETA * m, (1, 1))


def _make_sc_gather(rows, K, D):
    info = plsc.get_sparse_core_info()
    nworkers = info.num_cores * info.num_subcores
    per = rows // nworkers
    chunk = 128
    nchunks = per // chunk
    mesh = plsc.VectorSubcoreMesh(core_axis_name="c", subcore_axis_name="s")

    @functools.partial(
        pl.kernel, mesh=mesh,
        out_type=[
            jax.ShapeDtypeStruct((rows, D), jnp.float32),
            jax.ShapeDtypeStruct((nworkers, K), jnp.float32),
        ],
        scratch_types=[
            pltpu.VMEM((per,), jnp.int32),
            pltpu.VMEM((per, D), jnp.float32),
            pltpu.VMEM((K,), jnp.float32),
            pltpu.SemaphoreType.DMA,
        ],
        compiler_params=pltpu.CompilerParams(
            needs_layout_passes=False, use_tc_tiling_on_sc=False),
    )
    def sc_gather(emb_hbm, idx_hbm, zq_hbm, hist_hbm,
                  idx_v, rows_v, hist_v, sem):
        wid = lax.axis_index("s") * info.num_cores + lax.axis_index("c")
        base = wid * per
        pltpu.sync_copy(idx_hbm.at[pl.ds(base, per)], idx_v)
        copies = []
        for c in range(nchunks):
            copies.append(pltpu.async_copy(
                emb_hbm.at[idx_v.at[pl.ds(c * chunk, chunk)]],
                rows_v.at[pl.ds(c * chunk, chunk)], sem))
        for cp in copies:
            cp.wait()
        pltpu.sync_copy(rows_v, zq_hbm.at[pl.ds(base, per)])

        zeros16 = jnp.zeros((16,), jnp.float32)
        for j in range(K // 16):
            hist_v[pl.ds(j * 16, 16)] = zeros16
        ones16 = jnp.ones((16,), jnp.float32)
        for j in range(per // 16):
            ivec = idx_v[pl.ds(j * 16, 16)]
            plsc.addupdate_scatter(hist_v, [ivec], ones16)
        pltpu.sync_copy(hist_v, hist_hbm.at[wid])

    return sc_gather


def _finish_kernel(z_ref, zqf_ref, hist_ref, zq_ref, perp_ref):
    i = pl.program_id(0)
    nsteps = pl.num_programs(0)
    z_n = z_ref[0]
    zqt = zqf_ref[...].T

    zq_ref[0] = z_n + (zqt - z_n)

    @pl.when(i == nsteps - 1)
    def _():
        T = z_n.shape[1]
        counts = jnp.sum(hist_ref[...], axis=0, keepdims=True)
        e_mean = counts / (nsteps * T)
        plogp = e_mean * jnp.log(e_mean + 1e-10)
        perp_ref[...] = jnp.reshape(jnp.exp(-jnp.sum(plogp)), (1, 1))


def kernel(z, emb):
    N, W, T = z.shape
    K = emb.shape[0]
    rows = N * T
    zpsq = jnp.sum(z ** 2, axis=1).reshape(-1, 1)
    embsq = jnp.sum(emb ** 2, axis=1)[None, :]

    idx3, loss = pl.pallas_call(
        _dist_kernel,
        grid=(N,),
        in_specs=[
            pl.BlockSpec((1, W, T), lambda i: (i, 0, 0)),
            pl.BlockSpec((K, W), lambda i: (0, 0)),
            pl.BlockSpec((T, 1), lambda i: (i, 0)),
            pl.BlockSpec((1, K), lambda i: (0, 0)),
        ],
        out_specs=[
            pl.BlockSpec((1, 1, T), lambda i: (i, 0, 0)),
            pl.BlockSpec((1, 1), lambda i: (0, 0)),
        ],
        out_shape=[
            jax.ShapeDtypeStruct((N, 1, T), jnp.int32),
            jax.ShapeDtypeStruct((1, 1), jnp.float32),
        ],
        scratch_shapes=[
            pltpu.SMEM((1, 1), jnp.float32),
        ],
        compiler_params=pltpu.CompilerParams(
            dimension_semantics=("arbitrary",)),
    )(z, emb, zpsq, embsq)

    idx_flat = idx3.reshape(rows)
    zq_flat, hist = _make_sc_gather(rows, K, W)(emb, idx_flat)

    zq, perp = pl.pallas_call(
        _finish_kernel,
        grid=(N,),
        in_specs=[
            pl.BlockSpec((1, W, T), lambda i: (i, 0, 0)),
            pl.BlockSpec((T, W), lambda i: (i, 0)),
            pl.BlockSpec((32, K), lambda i: (0, 0)),
        ],
        out_specs=[
            pl.BlockSpec((1, W, T), lambda i: (i, 0, 0)),
            pl.BlockSpec((1, 1), lambda i: (0, 0)),
        ],
        out_shape=[
            jax.ShapeDtypeStruct((N, W, T), jnp.float32),
            jax.ShapeDtypeStruct((1, 1), jnp.float32),
        ],
        compiler_params=pltpu.CompilerParams(
            dimension_semantics=("arbitrary",)),
    )(z, zq_flat, hist)

    return zq, loss[0, 0], perp[0, 0]


# speedup vs baseline: 1.9249x; 1.9249x over previous
import jax
import jax.numpy as jnp
from jax.experimental import pallas as pl
from jax.experimental.pallas import tpu as pltpu

N_CODES = 1024
EDIM = 64
BETA = 0.25


def _vq_kernel(z_ref, emb_ref, emb2_ref, zpsq_ref, embsq_ref,
               zq_ref, loss_ref, perp_ref, counts_ref, acc_ref):
    i = pl.program_id(0)
    nsteps = pl.num_programs(0)
    z_n = z_ref[0]
    emb = emb_ref[...]
    T = z_n.shape[1]
    K = emb.shape[0]

    zp = z_n.T

    dot2 = jax.lax.dot_general(zp, emb2_ref[...],
                               (((1,), (1,)), ((), ())))
    d = (zpsq_ref[...] + embsq_ref[...]) - dot2

    dmin = jnp.min(d, axis=1, keepdims=True)
    iota_k = jax.lax.broadcasted_iota(jnp.int32, (T, K), 1)
    idx = jnp.min(jnp.where(d == dmin, iota_k, K), axis=1)

    p = (iota_k == idx[:, None]).astype(jnp.float32)
    zqt = jax.lax.dot_general(emb, p, (((0,), (1,)), ((), ())))

    zq_ref[0] = z_n + (zqt - z_n)

    diff = zqt - z_n
    part = jnp.sum(diff * diff)
    ones_t = jnp.ones((1, T), jnp.float32)
    cnt = jax.lax.dot_general(ones_t, p, (((1,), (0,)), ((), ())))

    @pl.when(i == 0)
    def _():
        acc_ref[0, 0] = 0.0
        counts_ref[...] = jnp.zeros_like(counts_ref)

    acc_ref[0, 0] += part
    counts_ref[...] += cnt

    @pl.when(i == nsteps - 1)
    def _():
        total_rows = nsteps * T
        m = acc_ref[0, 0] / (total_rows * EDIM)
        loss_ref[...] = jnp.reshape(m + BETA * m, (1, 1))
        e_mean = counts_ref[...] / total_rows
        plogp = e_mean * jnp.log(e_mean + 1e-10)
        perp_ref[...] = jnp.reshape(jnp.exp(-jnp.sum(plogp)), (1, 1))


def kernel(z, emb):
    N, W, T = z.shape
    K = emb.shape[0]
    zpsq = jnp.sum(z ** 2, axis=1).reshape(-1, 1)
    embsq = jnp.sum(emb ** 2, axis=1)[None, :]
    emb2 = emb + emb
    zq, loss, perp = pl.pallas_call(
        _vq_kernel,
        grid=(N,),
        in_specs=[
            pl.BlockSpec((1, W, T), lambda i: (i, 0, 0)),
            pl.BlockSpec((K, W), lambda i: (0, 0)),
            pl.BlockSpec((K, W), lambda i: (0, 0)),
            pl.BlockSpec((T, 1), lambda i: (i, 0)),
            pl.BlockSpec((1, K), lambda i: (0, 0)),
        ],
        out_specs=[
            pl.BlockSpec((1, W, T), lambda i: (i, 0, 0)),
            pl.BlockSpec((1, 1), lambda i: (0, 0)),
            pl.BlockSpec((1, 1), lambda i: (0, 0)),
        ],
        out_shape=[
            jax.ShapeDtypeStruct((N, W, T), jnp.float32),
            jax.ShapeDtypeStruct((1, 1), jnp.float32),
            jax.ShapeDtypeStruct((1, 1), jnp.float32),
        ],
        scratch_shapes=[
            pltpu.VMEM((1, K), jnp.float32),
            pltpu.SMEM((1, 1), jnp.float32),
        ],
        compiler_params=pltpu.CompilerParams(
            dimension_semantics=("arbitrary",)),
    )(z, emb, emb2, zpsq, embsq)
    return zq, loss[0, 0], perp[0, 0]
